# baseline (speedup 1.0000x reference)
import functools

import jax
import jax.numpy as jnp
from jax import lax
from jax.experimental import pallas as pl
from jax.experimental.pallas import tpu as pltpu
from jax.experimental.pallas import tpu_sc as plsc

BATCH = 16384
DIM = 32
LANES = 16
NUM_USERS = 100000
NUM_MOVIES = 1000000


def _make_main_call():
    info = plsc.get_sparse_core_info()
    nc, ns = info.num_cores, info.num_subcores
    b_per_s = BATCH // ns
    n_feat = DIM // nc
    mesh = plsc.VectorSubcoreMesh(core_axis_name="c", subcore_axis_name="s")

    BLK = 128
    M_BLOCKS = NUM_MOVIES // BLK
    M_T = (M_BLOCKS // ns + 1) * BLK
    M_CL = M_BLOCKS * BLK - M_T
    M_TAIL = M_BLOCKS * BLK
    U_BLOCKS = NUM_USERS // BLK
    U_T = (U_BLOCKS // ns + 1) * BLK
    U_CL = U_BLOCKS * BLK - U_T
    U_TAIL = U_BLOCKS * BLK

    @functools.partial(
        pl.kernel,
        mesh=mesh,
        compiler_params=pltpu.CompilerParams(needs_layout_passes=False),
        out_type=jax.ShapeDtypeStruct((nc, BATCH), jnp.float32),
        scratch_types=[
            pltpu.VMEM_SHARED((NUM_MOVIES,), jnp.float32),
            pltpu.VMEM_SHARED((NUM_USERS,), jnp.float32),
            pltpu.VMEM((b_per_s,), jnp.int32),
            pltpu.VMEM((b_per_s,), jnp.int32),
            pltpu.VMEM((b_per_s,), jnp.float32),
            pltpu.VMEM((b_per_s,), jnp.float32),
            pltpu.VMEM((b_per_s,), jnp.float32),
            pltpu.VMEM((DIM,), jnp.float32),
            pltpu.VMEM((4 * 8 * 64,), jnp.float32),
            pltpu.VMEM((4 * 8 * 32,), jnp.float32),
            pltpu.SemaphoreType.DMA,
            pltpu.SemaphoreType.DMA,
        ],
    )
    def main_call(uidx_hbm, midx_hbm, utab_hbm, mtab_hbm, mtail_hbm,
                  utail_hbm, w_hbm, out_hbm, spm_m, spm_u, uidx_v, midx_v,
                  gu_v, gm_v, acc_v, w_v, mtail_v, utail_v, ssem, gsem):
        c = lax.axis_index("c")
        s = lax.axis_index("s")
        base = s * b_per_s

        pltpu.sync_copy(uidx_hbm.at[pl.ds(base, b_per_s)], uidx_v)
        pltpu.sync_copy(midx_hbm.at[pl.ds(base, b_per_s)], midx_v)
        pltpu.sync_copy(w_hbm, w_v)
        pltpu.sync_copy(mtail_hbm, mtail_v)
        pltpu.sync_copy(utail_hbm, utail_v)

        for k in range(b_per_s // LANES):
            sl = pl.ds(k * LANES, LANES)
            acc_v[sl] = jnp.zeros((LANES,), jnp.float32)

        w_half = lax.select(c == 0, w_v[pl.ds(0, LANES)],
                            w_v[pl.ds(LANES, LANES)])

        m_off = jnp.minimum(s * M_T, M_CL)
        u_off = jnp.minimum(s * U_T, U_CL)

        for q in range(n_feat):
            blk, f = divmod(q, 8)
            msrc = mtab_hbm.at[c * 2 + blk, f]
            usrc = utab_hbm.at[c * 2 + blk, f]
            pend = [
                pltpu.async_copy(msrc.at[pl.ds(m_off, M_T)],
                                 spm_m.at[pl.ds(m_off, M_T)], ssem),
                pltpu.async_copy(usrc.at[pl.ds(u_off, U_T)],
                                 spm_u.at[pl.ds(u_off, U_T)], ssem),
                pltpu.async_copy(
                    mtail_v.at[pl.ds((c * 2 + blk) * 8 * 64 + f * 64, 64)],
                    spm_m.at[pl.ds(M_TAIL, 64)], ssem),
                pltpu.async_copy(
                    utail_v.at[pl.ds((c * 2 + blk) * 8 * 32 + f * 32, 32)],
                    spm_u.at[pl.ds(U_TAIL, 32)], ssem),
            ]
            for cp in pend:
                cp.wait()
            plsc.subcore_barrier()

            gcps = [pltpu.async_copy(spm_m.at[midx_v], gm_v, gsem),
                    pltpu.async_copy(spm_u.at[uidx_v], gu_v, gsem)]
            for cp in gcps:
                cp.wait()

            wd = w_half[q]

            def body(k, _):
                sl = pl.ds(k * LANES, LANES)
                acc_v[sl] = acc_v[sl] + gu_v[sl] * gm_v[sl] * wd
                return 0

            lax.fori_loop(0, b_per_s // LANES, body, 0)
            plsc.subcore_barrier()

        pltpu.sync_copy(acc_v, out_hbm.at[c, pl.ds(base, b_per_s)])

    return main_call


def _make_combine_call():
    info = plsc.get_sparse_core_info()
    num_workers = info.num_cores * info.num_subcores
    b_per_w = BATCH // num_workers
    mesh = plsc.VectorSubcoreMesh(core_axis_name="c", subcore_axis_name="s")

    @functools.partial(
        pl.kernel,
        mesh=mesh,
        compiler_params=pltpu.CompilerParams(needs_layout_passes=False),
        out_type=jax.ShapeDtypeStruct((BATCH,), jnp.float32),
        scratch_types=[
            pltpu.VMEM((b_per_w,), jnp.float32),
            pltpu.VMEM((b_per_w,), jnp.float32),
            pltpu.VMEM((b_per_w,), jnp.float32),
            pltpu.VMEM((LANES,), jnp.float32),
        ],
    )
    def combine_call(part_hbm, b_hbm, out_hbm, p0_v, p1_v, o_v, b_v):
        wid = lax.axis_index("s") * info.num_cores + lax.axis_index("c")
        base = wid * b_per_w
        pltpu.sync_copy(part_hbm.at[0, pl.ds(base, b_per_w)], p0_v)
        pltpu.sync_copy(part_hbm.at[1, pl.ds(base, b_per_w)], p1_v)
        pltpu.sync_copy(b_hbm, b_v)
        bias = b_v[pl.ds(0, LANES)]
        for k in range(b_per_w // LANES):
            sl = pl.ds(k * LANES, LANES)
            o_v[sl] = p0_v[sl] + p1_v[sl] + bias
        pltpu.sync_copy(o_v, out_hbm.at[pl.ds(base, b_per_w)])

    return combine_call


_MAIN_CALL = None
_COMBINE_CALL = None


def kernel(user_indices, movie_indices, user_table, movie_table, W, b):
    global _MAIN_CALL, _COMBINE_CALL
    if _MAIN_CALL is None:
        _MAIN_CALL = _make_main_call()
        _COMBINE_CALL = _make_combine_call()
    uidx = user_indices.astype(jnp.int32)
    midx = movie_indices.astype(jnp.int32)
    ut3 = user_table.T.reshape(4, 8, NUM_USERS)
    mt3 = movie_table.T.reshape(4, 8, NUM_MOVIES)
    mtail = mt3[:, :, 999936:].reshape(-1)
    utail = ut3[:, :, 99968:].reshape(-1)
    w_flat = W.reshape(DIM)
    b_vec = jnp.broadcast_to(b.reshape(()), (LANES,))
    parts = _MAIN_CALL(uidx, midx, ut3, mt3, mtail, utail, w_flat)
    out = _COMBINE_CALL(parts, b_vec)
    return out.reshape(BATCH, 1)


# speedup vs baseline: 1.0007x; 1.0007x over previous
import functools

import jax
import jax.numpy as jnp
from jax import lax
from jax.experimental import pallas as pl
from jax.experimental.pallas import tpu as pltpu
from jax.experimental.pallas import tpu_sc as plsc

BATCH = 16384
DIM = 32
LANES = 16
NUM_USERS = 100000
NUM_MOVIES = 1000000


def _make_main_call():
    info = plsc.get_sparse_core_info()
    nc, ns = info.num_cores, info.num_subcores
    b_per_s = BATCH // ns
    n_feat = DIM // nc
    mesh = plsc.VectorSubcoreMesh(core_axis_name="c", subcore_axis_name="s")

    BLK = 128
    M_BLOCKS = NUM_MOVIES // BLK
    M_T = (M_BLOCKS // ns + 1) * BLK
    M_CL = M_BLOCKS * BLK - M_T
    M_TAIL = M_BLOCKS * BLK
    U_BLOCKS = NUM_USERS // BLK
    U_T = (U_BLOCKS // ns + 1) * BLK
    U_CL = U_BLOCKS * BLK - U_T
    U_TAIL = U_BLOCKS * BLK

    @functools.partial(
        pl.kernel,
        mesh=mesh,
        compiler_params=pltpu.CompilerParams(needs_layout_passes=False),
        out_type=jax.ShapeDtypeStruct((nc, BATCH), jnp.float32),
        scratch_types=[
            pltpu.VMEM_SHARED((NUM_MOVIES,), jnp.float32),
            pltpu.VMEM_SHARED((NUM_USERS,), jnp.float32),
            pltpu.VMEM((b_per_s,), jnp.int32),
            pltpu.VMEM((b_per_s,), jnp.int32),
            pltpu.VMEM((b_per_s,), jnp.float32),
            pltpu.VMEM((b_per_s,), jnp.float32),
            pltpu.VMEM((b_per_s,), jnp.float32),
            pltpu.VMEM((DIM,), jnp.float32),
            pltpu.VMEM((4 * 8 * 64,), jnp.float32),
            pltpu.VMEM((4 * 8 * 32,), jnp.float32),
            pltpu.SemaphoreType.DMA,
            pltpu.SemaphoreType.DMA,
        ],
    )
    def main_call(uidx_hbm, midx_hbm, utab_hbm, mtab_hbm, mtail_hbm,
                  utail_hbm, w_hbm, out_hbm, spm_m, spm_u, uidx_v, midx_v,
                  gu_v, gm_v, acc_v, w_v, mtail_v, utail_v, ssem, gsem):
        c = lax.axis_index("c")
        s = lax.axis_index("s")
        base = s * b_per_s

        pltpu.sync_copy(uidx_hbm.at[pl.ds(base, b_per_s)], uidx_v)
        pltpu.sync_copy(midx_hbm.at[pl.ds(base, b_per_s)], midx_v)
        pltpu.sync_copy(w_hbm, w_v)
        pltpu.sync_copy(mtail_hbm, mtail_v)
        pltpu.sync_copy(utail_hbm, utail_v)

        for k in range(b_per_s // LANES):
            sl = pl.ds(k * LANES, LANES)
            acc_v[sl] = jnp.zeros((LANES,), jnp.float32)

        w_half = lax.select(c == 0, w_v[pl.ds(0, LANES)],
                            w_v[pl.ds(LANES, LANES)])

        m_off = jnp.minimum(s * M_T, M_CL)
        u_off = jnp.minimum(s * U_T, U_CL)

        for q in range(n_feat):
            blk, f = divmod(q, 8)
            msrc = mtab_hbm.at[c * 2 + blk, f]
            usrc = utab_hbm.at[c * 2 + blk, f]
            half = M_T // 2 // BLK * BLK
            rest = M_T - half
            pend = [
                pltpu.async_copy(msrc.at[pl.ds(m_off, half)],
                                 spm_m.at[pl.ds(m_off, half)], ssem),
                pltpu.async_copy(msrc.at[pl.ds(m_off + half, rest)],
                                 spm_m.at[pl.ds(m_off + half, rest)], ssem),
                pltpu.async_copy(usrc.at[pl.ds(u_off, U_T)],
                                 spm_u.at[pl.ds(u_off, U_T)], ssem),
                pltpu.async_copy(
                    mtail_v.at[pl.ds((c * 2 + blk) * 8 * 64 + f * 64, 64)],
                    spm_m.at[pl.ds(M_TAIL, 64)], ssem),
                pltpu.async_copy(
                    utail_v.at[pl.ds((c * 2 + blk) * 8 * 32 + f * 32, 32)],
                    spm_u.at[pl.ds(U_TAIL, 32)], ssem),
            ]
            for cp in pend:
                cp.wait()
            plsc.subcore_barrier()

            gcps = [pltpu.async_copy(spm_m.at[midx_v], gm_v, gsem),
                    pltpu.async_copy(spm_u.at[uidx_v], gu_v, gsem)]
            for cp in gcps:
                cp.wait()

            wd = w_half[q]

            def body(k, _):
                sl = pl.ds(k * LANES, LANES)
                acc_v[sl] = acc_v[sl] + gu_v[sl] * gm_v[sl] * wd
                return 0

            lax.fori_loop(0, b_per_s // LANES, body, 0)
            plsc.subcore_barrier()

        pltpu.sync_copy(acc_v, out_hbm.at[c, pl.ds(base, b_per_s)])

    return main_call


def _make_combine_call():
    info = plsc.get_sparse_core_info()
    num_workers = info.num_cores * info.num_subcores
    b_per_w = BATCH // num_workers
    mesh = plsc.VectorSubcoreMesh(core_axis_name="c", subcore_axis_name="s")

    @functools.partial(
        pl.kernel,
        mesh=mesh,
        compiler_params=pltpu.CompilerParams(needs_layout_passes=False),
        out_type=jax.ShapeDtypeStruct((BATCH,), jnp.float32),
        scratch_types=[
            pltpu.VMEM((b_per_w,), jnp.float32),
            pltpu.VMEM((b_per_w,), jnp.float32),
            pltpu.VMEM((b_per_w,), jnp.float32),
            pltpu.VMEM((LANES,), jnp.float32),
        ],
    )
    def combine_call(part_hbm, b_hbm, out_hbm, p0_v, p1_v, o_v, b_v):
        wid = lax.axis_index("s") * info.num_cores + lax.axis_index("c")
        base = wid * b_per_w
        pltpu.sync_copy(part_hbm.at[0, pl.ds(base, b_per_w)], p0_v)
        pltpu.sync_copy(part_hbm.at[1, pl.ds(base, b_per_w)], p1_v)
        pltpu.sync_copy(b_hbm, b_v)
        bias = b_v[pl.ds(0, LANES)]
        for k in range(b_per_w // LANES):
            sl = pl.ds(k * LANES, LANES)
            o_v[sl] = p0_v[sl] + p1_v[sl] + bias
        pltpu.sync_copy(o_v, out_hbm.at[pl.ds(base, b_per_w)])

    return combine_call


_MAIN_CALL = None
_COMBINE_CALL = None


def kernel(user_indices, movie_indices, user_table, movie_table, W, b):
    global _MAIN_CALL, _COMBINE_CALL
    if _MAIN_CALL is None:
        _MAIN_CALL = _make_main_call()
        _COMBINE_CALL = _make_combine_call()
    uidx = user_indices.astype(jnp.int32)
    midx = movie_indices.astype(jnp.int32)
    ut3 = user_table.T.reshape(4, 8, NUM_USERS)
    mt3 = movie_table.T.reshape(4, 8, NUM_MOVIES)
    mtail = mt3[:, :, 999936:].reshape(-1)
    utail = ut3[:, :, 99968:].reshape(-1)
    w_flat = W.reshape(DIM)
    b_vec = jnp.broadcast_to(b.reshape(()), (LANES,))
    parts = _MAIN_CALL(uidx, midx, ut3, mt3, mtail, utail, w_flat)
    out = _COMBINE_CALL(parts, b_vec)
    return out.reshape(BATCH, 1)


# speedup vs baseline: 1.0016x; 1.0009x over previous
import functools

import jax
import jax.numpy as jnp
from jax import lax
from jax.experimental import pallas as pl
from jax.experimental.pallas import tpu as pltpu
from jax.experimental.pallas import tpu_sc as plsc

BATCH = 16384
DIM = 32
LANES = 16
NUM_USERS = 100000
NUM_MOVIES = 1000000


def _make_main_call():
    info = plsc.get_sparse_core_info()
    nc, ns = info.num_cores, info.num_subcores
    b_per_s = BATCH // ns
    n_feat = DIM // nc
    mesh = plsc.VectorSubcoreMesh(core_axis_name="c", subcore_axis_name="s")

    BLK = 128
    M_BLOCKS = NUM_MOVIES // BLK
    M_T = (M_BLOCKS // ns + 1) * BLK
    M_CL = M_BLOCKS * BLK - M_T
    M_TAIL = M_BLOCKS * BLK
    U_BLOCKS = NUM_USERS // BLK
    U_T = (U_BLOCKS // ns + 1) * BLK
    U_CL = U_BLOCKS * BLK - U_T
    U_TAIL = U_BLOCKS * BLK

    @functools.partial(
        pl.kernel,
        mesh=mesh,
        compiler_params=pltpu.CompilerParams(needs_layout_passes=False),
        out_type=jax.ShapeDtypeStruct((nc, BATCH), jnp.float32),
        scratch_types=[
            pltpu.VMEM_SHARED((NUM_MOVIES,), jnp.float32),
            pltpu.VMEM_SHARED((NUM_USERS,), jnp.float32),
            pltpu.VMEM((b_per_s,), jnp.int32),
            pltpu.VMEM((b_per_s,), jnp.int32),
            pltpu.VMEM((b_per_s,), jnp.float32),
            pltpu.VMEM((b_per_s,), jnp.float32),
            pltpu.VMEM((b_per_s,), jnp.float32),
            pltpu.VMEM((DIM,), jnp.float32),
            pltpu.VMEM((4 * 8 * 64,), jnp.float32),
            pltpu.VMEM((4 * 8 * 32,), jnp.float32),
            pltpu.SemaphoreType.DMA,
            pltpu.SemaphoreType.DMA,
        ],
    )
    def main_call(uidx_hbm, midx_hbm, utab_hbm, mtab_hbm, mtail_hbm,
                  utail_hbm, w_hbm, out_hbm, spm_m, spm_u, uidx_v, midx_v,
                  gu_v, gm_v, acc_v, w_v, mtail_v, utail_v, ssem, gsem):
        c = lax.axis_index("c")
        s = lax.axis_index("s")
        base = s * b_per_s

        pltpu.sync_copy(uidx_hbm.at[pl.ds(base, b_per_s)], uidx_v)
        pltpu.sync_copy(midx_hbm.at[pl.ds(base, b_per_s)], midx_v)
        pltpu.sync_copy(w_hbm, w_v)
        pltpu.sync_copy(mtail_hbm, mtail_v)
        pltpu.sync_copy(utail_hbm, utail_v)

        for k in range(b_per_s // LANES):
            sl = pl.ds(k * LANES, LANES)
            acc_v[sl] = jnp.zeros((LANES,), jnp.float32)

        w_half = lax.select(c == 0, w_v[pl.ds(0, LANES)],
                            w_v[pl.ds(LANES, LANES)])

        m_off = jnp.minimum(s * M_T, M_CL)
        u_off = jnp.minimum(s * U_T, U_CL)

        for q in range(n_feat):
            blk, f = divmod(q, 8)
            msrc = mtab_hbm.at[c * 2 + blk, f]
            usrc = utab_hbm.at[c * 2 + blk, f]
            pend = [
                pltpu.async_copy(msrc.at[pl.ds(m_off, M_T)],
                                 spm_m.at[pl.ds(m_off, M_T)], ssem),
                pltpu.async_copy(usrc.at[pl.ds(u_off, U_T)],
                                 spm_u.at[pl.ds(u_off, U_T)], ssem),
                pltpu.async_copy(
                    mtail_v.at[pl.ds((c * 2 + blk) * 8 * 64 + f * 64, 64)],
                    spm_m.at[pl.ds(M_TAIL, 64)], ssem),
                pltpu.async_copy(
                    utail_v.at[pl.ds((c * 2 + blk) * 8 * 32 + f * 32, 32)],
                    spm_u.at[pl.ds(U_TAIL, 32)], ssem),
            ]
            for cp in pend:
                cp.wait()
            plsc.subcore_barrier()

            gcps = [pltpu.async_copy(spm_m.at[midx_v], gm_v, gsem),
                    pltpu.async_copy(spm_u.at[uidx_v], gu_v, gsem)]
            for cp in gcps:
                cp.wait()

            wd = w_half[q]

            def body(k, _):
                sl = pl.ds(k * LANES, LANES)
                acc_v[sl] = acc_v[sl] + gu_v[sl] * gm_v[sl] * wd
                return 0

            lax.fori_loop(0, b_per_s // LANES, body, 0)
            plsc.subcore_barrier()

        pltpu.sync_copy(acc_v, out_hbm.at[c, pl.ds(base, b_per_s)])

    return main_call


def _make_combine_call():
    info = plsc.get_sparse_core_info()
    num_workers = info.num_cores * info.num_subcores
    b_per_w = BATCH // num_workers
    mesh = plsc.VectorSubcoreMesh(core_axis_name="c", subcore_axis_name="s")

    @functools.partial(
        pl.kernel,
        mesh=mesh,
        compiler_params=pltpu.CompilerParams(needs_layout_passes=False),
        out_type=jax.ShapeDtypeStruct((BATCH,), jnp.float32),
        scratch_types=[
            pltpu.VMEM((b_per_w,), jnp.float32),
            pltpu.VMEM((b_per_w,), jnp.float32),
            pltpu.VMEM((b_per_w,), jnp.float32),
            pltpu.VMEM((LANES,), jnp.float32),
        ],
    )
    def combine_call(part_hbm, b_hbm, out_hbm, p0_v, p1_v, o_v, b_v):
        wid = lax.axis_index("s") * info.num_cores + lax.axis_index("c")
        base = wid * b_per_w
        pltpu.sync_copy(part_hbm.at[0, pl.ds(base, b_per_w)], p0_v)
        pltpu.sync_copy(part_hbm.at[1, pl.ds(base, b_per_w)], p1_v)
        pltpu.sync_copy(b_hbm, b_v)
        bias = b_v[pl.ds(0, LANES)]
        for k in range(b_per_w // LANES):
            sl = pl.ds(k * LANES, LANES)
            o_v[sl] = p0_v[sl] + p1_v[sl] + bias
        pltpu.sync_copy(o_v, out_hbm.at[pl.ds(base, b_per_w)])

    return combine_call


_MAIN_CALL = None
_COMBINE_CALL = None


def kernel(user_indices, movie_indices, user_table, movie_table, W, b):
    global _MAIN_CALL, _COMBINE_CALL
    if _MAIN_CALL is None:
        _MAIN_CALL = _make_main_call()
        _COMBINE_CALL = _make_combine_call()
    uidx = user_indices.astype(jnp.int32)
    midx = movie_indices.astype(jnp.int32)
    ut3 = user_table.T.reshape(4, 8, NUM_USERS)
    mt3 = movie_table.T.reshape(4, 8, NUM_MOVIES)
    mtail = mt3[:, :, 999936:].reshape(-1)
    utail = ut3[:, :, 99968:].reshape(-1)
    w_flat = W.reshape(DIM)
    b_vec = jnp.broadcast_to(b.reshape(()), (LANES,))
    parts = _MAIN_CALL(uidx, midx, ut3, mt3, mtail, utail, w_flat)
    out = _COMBINE_CALL(parts, b_vec)
    return out.reshape(BATCH, 1)


# speedup vs baseline: 1.0328x; 1.0311x over previous
import functools

import jax
import jax.numpy as jnp
from jax import lax
from jax.experimental import pallas as pl
from jax.experimental.pallas import tpu as pltpu
from jax.experimental.pallas import tpu_sc as plsc

BATCH = 16384
DIM = 32
LANES = 16
NUM_USERS = 100000
NUM_MOVIES = 1000000


def _make_main_call():
    info = plsc.get_sparse_core_info()
    nc, ns = info.num_cores, info.num_subcores
    b_per_s = BATCH // ns
    n_feat = DIM // nc
    mesh = plsc.VectorSubcoreMesh(core_axis_name="c", subcore_axis_name="s")

    BLK = 128
    M_BLOCKS = NUM_MOVIES // BLK
    M_T = (M_BLOCKS // ns + 1) * BLK
    M_CL = M_BLOCKS * BLK - M_T
    M_TAIL = M_BLOCKS * BLK
    U_BLOCKS = NUM_USERS // BLK
    U_T = (U_BLOCKS // ns + 1) * BLK
    U_CL = U_BLOCKS * BLK - U_T
    U_TAIL = U_BLOCKS * BLK

    @functools.partial(
        pl.kernel,
        mesh=mesh,
        compiler_params=pltpu.CompilerParams(needs_layout_passes=False),
        out_type=jax.ShapeDtypeStruct((nc, BATCH), jnp.float32),
        scratch_types=[
            pltpu.VMEM_SHARED((NUM_MOVIES,), jnp.float32),
            pltpu.VMEM_SHARED((NUM_USERS,), jnp.float32),
            pltpu.VMEM((b_per_s,), jnp.int32),
            pltpu.VMEM((b_per_s,), jnp.int32),
            pltpu.VMEM((b_per_s,), jnp.float32),
            pltpu.VMEM((b_per_s,), jnp.float32),
            pltpu.VMEM((b_per_s,), jnp.float32),
            pltpu.VMEM((DIM,), jnp.float32),
            pltpu.VMEM((4 * 8 * 64,), jnp.float32),
            pltpu.VMEM((4 * 8 * 32,), jnp.float32),
            pltpu.SemaphoreType.DMA,
            pltpu.SemaphoreType.DMA,
        ],
    )
    def main_call(uidx_hbm, midx_hbm, utab_hbm, mtab_hbm, mtail_hbm,
                  utail_hbm, w_hbm, out_hbm, spm_m, spm_u, uidx_v, midx_v,
                  gu_v, gm_v, acc_v, w_v, mtail_v, utail_v, ssem, gsem):
        c = lax.axis_index("c")
        s = lax.axis_index("s")
        base = s * b_per_s

        pltpu.sync_copy(uidx_hbm.at[pl.ds(base, b_per_s)], uidx_v)
        pltpu.sync_copy(midx_hbm.at[pl.ds(base, b_per_s)], midx_v)
        pltpu.sync_copy(w_hbm, w_v)
        pltpu.sync_copy(mtail_hbm, mtail_v)
        pltpu.sync_copy(utail_hbm, utail_v)

        for k in range(b_per_s // LANES):
            sl = pl.ds(k * LANES, LANES)
            acc_v[sl] = jnp.zeros((LANES,), jnp.float32)

        w_half = lax.select(c == 0, w_v[pl.ds(0, LANES)],
                            w_v[pl.ds(LANES, LANES)])

        m_off = jnp.minimum(s * M_T, M_CL)
        u_off = jnp.minimum(s * U_T, U_CL)

        def stage(q):
            blk, f = divmod(q, 8)
            msrc = mtab_hbm.at[c * 2 + blk, f]
            usrc = utab_hbm.at[c * 2 + blk, f]
            return [
                pltpu.async_copy(msrc.at[pl.ds(m_off, M_T)],
                                 spm_m.at[pl.ds(m_off, M_T)], ssem),
                pltpu.async_copy(usrc.at[pl.ds(u_off, U_T)],
                                 spm_u.at[pl.ds(u_off, U_T)], ssem),
                pltpu.async_copy(
                    mtail_v.at[pl.ds((c * 2 + blk) * 8 * 64 + f * 64, 64)],
                    spm_m.at[pl.ds(M_TAIL, 64)], ssem),
                pltpu.async_copy(
                    utail_v.at[pl.ds((c * 2 + blk) * 8 * 32 + f * 32, 32)],
                    spm_u.at[pl.ds(U_TAIL, 32)], ssem),
            ]

        pend = stage(0)
        for q in range(n_feat):
            for cp in pend:
                cp.wait()
            plsc.subcore_barrier()

            gcps = [pltpu.async_copy(spm_m.at[midx_v], gm_v, gsem),
                    pltpu.async_copy(spm_u.at[uidx_v], gu_v, gsem)]
            for cp in gcps:
                cp.wait()
            plsc.subcore_barrier()

            pend = stage(q + 1) if q + 1 < n_feat else []

            wd = w_half[q]

            def body(k, _):
                sl = pl.ds(k * LANES, LANES)
                acc_v[sl] = acc_v[sl] + gu_v[sl] * gm_v[sl] * wd
                return 0

            lax.fori_loop(0, b_per_s // LANES, body, 0)

        pltpu.sync_copy(acc_v, out_hbm.at[c, pl.ds(base, b_per_s)])

    return main_call


def _make_combine_call():
    info = plsc.get_sparse_core_info()
    num_workers = info.num_cores * info.num_subcores
    b_per_w = BATCH // num_workers
    mesh = plsc.VectorSubcoreMesh(core_axis_name="c", subcore_axis_name="s")

    @functools.partial(
        pl.kernel,
        mesh=mesh,
        compiler_params=pltpu.CompilerParams(needs_layout_passes=False),
        out_type=jax.ShapeDtypeStruct((BATCH,), jnp.float32),
        scratch_types=[
            pltpu.VMEM((b_per_w,), jnp.float32),
            pltpu.VMEM((b_per_w,), jnp.float32),
            pltpu.VMEM((b_per_w,), jnp.float32),
            pltpu.VMEM((LANES,), jnp.float32),
        ],
    )
    def combine_call(part_hbm, b_hbm, out_hbm, p0_v, p1_v, o_v, b_v):
        wid = lax.axis_index("s") * info.num_cores + lax.axis_index("c")
        base = wid * b_per_w
        pltpu.sync_copy(part_hbm.at[0, pl.ds(base, b_per_w)], p0_v)
        pltpu.sync_copy(part_hbm.at[1, pl.ds(base, b_per_w)], p1_v)
        pltpu.sync_copy(b_hbm, b_v)
        bias = b_v[pl.ds(0, LANES)]
        for k in range(b_per_w // LANES):
            sl = pl.ds(k * LANES, LANES)
            o_v[sl] = p0_v[sl] + p1_v[sl] + bias
        pltpu.sync_copy(o_v, out_hbm.at[pl.ds(base, b_per_w)])

    return combine_call


_MAIN_CALL = None
_COMBINE_CALL = None


def kernel(user_indices, movie_indices, user_table, movie_table, W, b):
    global _MAIN_CALL, _COMBINE_CALL
    if _MAIN_CALL is None:
        _MAIN_CALL = _make_main_call()
        _COMBINE_CALL = _make_combine_call()
    uidx = user_indices.astype(jnp.int32)
    midx = movie_indices.astype(jnp.int32)
    ut3 = user_table.T.reshape(4, 8, NUM_USERS)
    mt3 = movie_table.T.reshape(4, 8, NUM_MOVIES)
    mtail = mt3[:, :, 999936:].reshape(-1)
    utail = ut3[:, :, 99968:].reshape(-1)
    w_flat = W.reshape(DIM)
    b_vec = jnp.broadcast_to(b.reshape(()), (LANES,))
    parts = _MAIN_CALL(uidx, midx, ut3, mt3, mtail, utail, w_flat)
    out = _COMBINE_CALL(parts, b_vec)
    return out.reshape(BATCH, 1)
